# baseline (device time: 168672 ns/iter reference)
import jax
import jax.numpy as jnp
from jax import lax
from jax.experimental import pallas as pl
from jax.experimental.pallas import tpu as pltpu

M_PER = 2048
D_DIM = 2048
F_DIM = 8192
Q = 512
TF = 256
N_TF = F_DIM // TF
XC = 256
NC = 16
CF = F_DIM // NC
TPC = CF // TF
COMM = True


def kernel(x, dy):
    def body(x_hbm, dy_hbm, out_hbm,
             xstage, xall, dystage, e_buf, d_buf, r1_buf, r2_buf, out_stage,
             local_sems, out_sem, y_send_sems, y_recv_sems,
             z_send_sems, z_recv_sems,
             xa_send_sems, xa_recv_sems, xb_send_sems, xb_recv_sems):
        my_x = lax.axis_index("x")
        my_y = lax.axis_index("y")
        my_z = lax.axis_index("z")
        y_nbr = (my_x, 1 - my_y, my_z)
        z_nbr = (my_x, my_y, 1 - my_z)
        x_nbr = (1 - my_x, my_y, my_z)

        x_is_0 = (my_x == 0)

        def in_y(c):
            return jnp.where(x_is_0, c <= 10, c >= 5)

        def excl(c):
            return jnp.where(x_is_0, c < 5, c >= 11)

        if COMM:
            barrier_sem = pltpu.get_barrier_semaphore()
            for nbr in (y_nbr, z_nbr, x_nbr):
                pl.semaphore_signal(barrier_sem, inc=1, device_id=nbr,
                                    device_id_type=pl.DeviceIdType.MESH)
            pl.semaphore_wait(barrier_sem, 3)

        def y_rdma(i):
            return pltpu.make_async_remote_copy(
                src_ref=e_buf.at[i], dst_ref=r1_buf.at[i],
                send_sem=y_send_sems.at[i], recv_sem=y_recv_sems.at[i],
                device_id=y_nbr, device_id_type=pl.DeviceIdType.MESH)

        def z_rdma(i):
            return pltpu.make_async_remote_copy(
                src_ref=r1_buf.at[i], dst_ref=r2_buf.at[i],
                send_sem=z_send_sems.at[i], recv_sem=z_recv_sems.at[i],
                device_id=z_nbr, device_id_type=pl.DeviceIdType.MESH)

        def xa_rdma(i):
            return pltpu.make_async_remote_copy(
                src_ref=r1_buf.at[i], dst_ref=r1_buf.at[i],
                send_sem=xa_send_sems.at[i], recv_sem=xa_recv_sems.at[i],
                device_id=x_nbr, device_id_type=pl.DeviceIdType.MESH)

        def xb_rdma(i):
            return pltpu.make_async_remote_copy(
                src_ref=r2_buf.at[i], dst_ref=r2_buf.at[i],
                send_sem=xb_send_sems.at[i], recv_sem=xb_recv_sems.at[i],
                device_id=x_nbr, device_id_type=pl.DeviceIdType.MESH)

        qe_col = (1 - my_y) * 1024 + my_z * Q
        qd_col = my_y * 1024
        xoffs = []
        for i, coloff in enumerate([qe_col, qd_col, qd_col + Q]):
            for c in range(Q // XC):
                xoffs.append((i * Q + c * XC, coloff + c * XC))
        cps = [None, None]
        for j, (dst_off, src_off) in enumerate(xoffs):
            slot = j % 2
            cps[slot] = pltpu.make_async_copy(
                x_hbm.at[:, pl.ds(src_off, XC)], xstage.at[slot],
                local_sems.at[slot])
            cps[slot].start()
            if j > 0:
                prev_dst = xoffs[j - 1][0]
                cps[(j - 1) % 2].wait()
                xall[:, prev_dst:prev_dst + XC] = (
                    xstage[(j - 1) % 2].astype(jnp.bfloat16))
        cps[(len(xoffs) - 1) % 2].wait()
        last_dst = xoffs[-1][0]
        xall[:, last_dst:last_dst + XC] = (
            xstage[(len(xoffs) - 1) % 2].astype(jnp.bfloat16))

        def dy_cp(t, slot):
            return pltpu.make_async_copy(
                dy_hbm.at[:, pl.ds(t * TF, TF)], dystage.at[slot],
                local_sems.at[slot])

        dy_cp(0, 0).start()

        z_is_0 = (my_z == 0)

        def emit_out(i):
            d = d_buf[:, pl.ds(i * CF, CF)].astype(jnp.float32)
            r1 = r1_buf[i].astype(jnp.float32)
            r2 = r2_buf[i].astype(jnp.float32)
            out_stage[0:Q, :] = (
                d[0:Q, :] + jnp.where(z_is_0, r1, r2)).astype(jnp.bfloat16)
            out_stage[Q:, :] = (
                d[Q:, :] + jnp.where(z_is_0, r2, r1)).astype(jnp.bfloat16)
            pltpu.make_async_copy(
                out_stage, out_hbm.at[:, pl.ds(i * CF, CF)], out_sem).start()

        def out_wait(i):
            pltpu.make_async_copy(
                out_stage, out_hbm.at[:, pl.ds(i * CF, CF)], out_sem).wait()

        L1 = 2
        L2 = 4
        L3 = 6

        def stage2(cz):
            @pl.when(in_y(cz))
            def _():
                y_rdma(cz).wait_recv()
                z_rdma(cz).start()

                @pl.when(excl(cz))
                def _():
                    xa_rdma(cz).start()

        def stage3(cb):
            @pl.when(in_y(cb))
            def _():
                z_rdma(cb).wait_recv()

                @pl.when(excl(cb))
                def _():
                    xb_rdma(cb).start()

            @pl.when(jnp.logical_not(in_y(cb)))
            def _():
                xa_rdma(cb).wait_recv()

        def stage4(ce):
            @pl.when(jnp.logical_not(in_y(ce)))
            def _():
                xb_rdma(ce).wait_recv()

            @pl.when(ce > 0)
            def _():
                out_wait(ce - 1)

            emit_out(ce)

        def chunk_end(ci):
            @pl.when(in_y(ci))
            def _():
                y_rdma(ci).start()

            @pl.when(ci >= L1)
            def _():
                stage2(ci - L1)

            @pl.when(ci >= L2)
            def _():
                stage3(ci - L2)

            @pl.when(ci >= L3)
            def _():
                stage4(ci - L3)

        def compute_step(t, carry):
            slot = t % 2

            @pl.when(t + 1 < N_TF)
            def _():
                dy_cp(t + 1, 1 - slot).start()

            dy_cp(t, slot).wait()
            dyt = dystage[slot].astype(jnp.bfloat16)
            res = lax.dot_general(
                xall[:, :], dyt,
                dimension_numbers=(((0,), (0,)), ((), ())),
                preferred_element_type=jnp.float32,
            )
            ci, cc = t // TPC, (t % TPC) * TF
            e_buf[ci, :, pl.ds(cc, TF)] = res[0:Q, :].astype(jnp.bfloat16)
            d_buf[:, pl.ds(t * TF, TF)] = res[Q:, :].astype(jnp.bfloat16)

            if COMM:
                @pl.when(t % TPC == TPC - 1)
                def _():
                    chunk_end(ci)

            return carry

        lax.fori_loop(0, N_TF, compute_step, 0)

        if COMM:
            def tail_step(k, carry):
                @pl.when(k - L1 < NC)
                def _():
                    stage2(k - L1)

                @pl.when(k - L2 < NC)
                def _():
                    stage3(k - L2)

                stage4(k - L3)
                return carry

            lax.fori_loop(NC, NC + L3, tail_step, 0)
        else:
            for i in range(NC):
                if i > 0:
                    out_wait(i - 1)
                emit_out(i)
        out_wait(NC - 1)

        if COMM:
            def drain_step(i, carry):
                @pl.when(in_y(i))
                def _():
                    y_rdma(i).wait_send()
                    z_rdma(i).wait_send()

                @pl.when(excl(i))
                def _():
                    xa_rdma(i).wait_send()
                    xb_rdma(i).wait_send()

                return carry

            lax.fori_loop(0, NC, drain_step, 0)

    return pl.pallas_call(
        body,
        out_shape=jax.ShapeDtypeStruct((D_DIM // 2, F_DIM), jnp.bfloat16),
        in_specs=[
            pl.BlockSpec(memory_space=pl.ANY),
            pl.BlockSpec(memory_space=pl.ANY),
        ],
        out_specs=pl.BlockSpec(memory_space=pl.ANY),
        scratch_shapes=[
            pltpu.VMEM((2, M_PER, XC), jnp.float32),
            pltpu.VMEM((M_PER, 3 * Q), jnp.bfloat16),
            pltpu.VMEM((2, M_PER, TF), jnp.float32),
            pltpu.VMEM((NC, Q, CF), jnp.bfloat16),
            pltpu.VMEM((2 * Q, F_DIM), jnp.bfloat16),
            pltpu.VMEM((NC, Q, CF), jnp.bfloat16),
            pltpu.VMEM((NC, Q, CF), jnp.bfloat16),
            pltpu.VMEM((2 * Q, CF), jnp.bfloat16),
            pltpu.SemaphoreType.DMA((2,)),
            pltpu.SemaphoreType.DMA,
            pltpu.SemaphoreType.DMA((NC,)),
            pltpu.SemaphoreType.DMA((NC,)),
            pltpu.SemaphoreType.DMA((NC,)),
            pltpu.SemaphoreType.DMA((NC,)),
            pltpu.SemaphoreType.DMA((NC,)),
            pltpu.SemaphoreType.DMA((NC,)),
            pltpu.SemaphoreType.DMA((NC,)),
            pltpu.SemaphoreType.DMA((NC,)),
        ],
        compiler_params=pltpu.CompilerParams(
            collective_id=0 if COMM else None,
            vmem_limit_bytes=64 * 1024 * 1024,
        ),
    )(x, dy)


# device time: 136993 ns/iter; 1.2312x vs baseline; 1.2312x over previous
import jax
import jax.numpy as jnp
from jax import lax
from jax.experimental import pallas as pl
from jax.experimental.pallas import tpu as pltpu

M_PER = 2048
D_DIM = 2048
F_DIM = 8192
Q = 512
TF = 256
N_TF = F_DIM // TF
XC = 256
NC = 16
CF = F_DIM // NC
TPC = CF // TF
COMM = True


def kernel(x, dy):
    def body(x_hbm, dy_hbm, out_hbm,
             xstage, xall, dystage, e_buf, d_buf, r1_buf, r2_buf, out_stage,
             local_sems, out_sem, y_send_sems, y_recv_sems,
             z_send_sems, z_recv_sems):
        my_x = lax.axis_index("x")
        my_y = lax.axis_index("y")
        my_z = lax.axis_index("z")
        y_nbr = (my_x, 1 - my_y, my_z)
        z_nbr = (my_x, my_y, 1 - my_z)

        if COMM:
            barrier_sem = pltpu.get_barrier_semaphore()
            for nbr in (y_nbr, z_nbr):
                pl.semaphore_signal(barrier_sem, inc=1, device_id=nbr,
                                    device_id_type=pl.DeviceIdType.MESH)
            pl.semaphore_wait(barrier_sem, 2)

        def y_rdma(i):
            return pltpu.make_async_remote_copy(
                src_ref=e_buf.at[i], dst_ref=r1_buf.at[i],
                send_sem=y_send_sems.at[i], recv_sem=y_recv_sems.at[i],
                device_id=y_nbr, device_id_type=pl.DeviceIdType.MESH)

        def z_rdma(i):
            return pltpu.make_async_remote_copy(
                src_ref=r1_buf.at[i], dst_ref=r2_buf.at[i],
                send_sem=z_send_sems.at[i], recv_sem=z_recv_sems.at[i],
                device_id=z_nbr, device_id_type=pl.DeviceIdType.MESH)

        def dy_cp(t, slot):
            return pltpu.make_async_copy(
                dy_hbm.at[:, pl.ds(t * TF, TF)], dystage.at[slot],
                local_sems.at[slot])

        def dy_cp_early(t):
            return pltpu.make_async_copy(
                dy_hbm.at[:, pl.ds(t * TF, TF)], dystage.at[t],
                local_sems.at[2 + t])

        dy_cp_early(0).start()
        dy_cp_early(1).start()

        qe_col = (1 - my_y) * 1024 + my_z * Q
        qd_col = my_y * 1024
        xoffs = [(0, qe_col), (XC, qe_col + XC)]
        for c in range(2 * Q // XC):
            xoffs.append((2 * XC + c * XC, qd_col + c * XC))

        def x_cp(j):
            return pltpu.make_async_copy(
                x_hbm.at[:, pl.ds(xoffs[j][1], XC)], xstage.at[j % 2],
                local_sems.at[j % 2])

        def load_x(lo, hi):
            for j in range(lo, hi):
                x_cp(j).start()
                if j > lo:
                    x_cp(j - 1).wait()
                    dst = xoffs[j - 1][0]
                    xall[:, dst:dst + XC] = xstage[(j - 1) % 2].astype(
                        jnp.bfloat16)
            x_cp(hi - 1).wait()
            dst = xoffs[hi - 1][0]
            xall[:, dst:dst + XC] = xstage[(hi - 1) % 2].astype(jnp.bfloat16)

        load_x(0, 2)

        if COMM:
            for t in range(2):
                dy_cp_early(t).wait()
                dyt = dystage[t].astype(jnp.bfloat16)
                e0 = lax.dot_general(
                    xall[:, 0:Q], dyt,
                    dimension_numbers=(((0,), (0,)), ((), ())),
                    preferred_element_type=jnp.float32,
                )
                e_buf[0, :, pl.ds(t * TF, TF)] = e0.astype(jnp.bfloat16)
            y_rdma(0).start()
        else:
            for t in range(2):
                dy_cp_early(t).wait()

        load_x(2, len(xoffs))

        dy_cp(0, 0).start()

        z_is_0 = (my_z == 0)

        def emit_out(i):
            d = d_buf[:, pl.ds(i * CF, CF)].astype(jnp.float32)
            r1 = r1_buf[i].astype(jnp.float32)
            r2 = r2_buf[i].astype(jnp.float32)
            out_stage[0:Q, :] = (
                d[0:Q, :] + jnp.where(z_is_0, r1, r2)).astype(jnp.bfloat16)
            out_stage[Q:, :] = (
                d[Q:, :] + jnp.where(z_is_0, r2, r1)).astype(jnp.bfloat16)
            pltpu.make_async_copy(
                out_stage, out_hbm.at[:, pl.ds(i * CF, CF)], out_sem).start()

        def out_wait(i):
            pltpu.make_async_copy(
                out_stage, out_hbm.at[:, pl.ds(i * CF, CF)], out_sem).wait()

        L1 = 1
        L2 = 3

        def chunk_end(ci):
            @pl.when(ci >= 1)
            def _():
                y_rdma(ci).start()

            @pl.when(ci >= L1)
            def _():
                cz = ci - L1
                y_rdma(cz).wait_recv()
                z_rdma(cz).start()

            @pl.when(ci >= L2)
            def _():
                ce = ci - L2
                z_rdma(ce).wait_recv()

                @pl.when(ce > 0)
                def _():
                    out_wait(ce - 1)

                emit_out(ce)

        def compute_step(t, carry):
            slot = t % 2

            @pl.when(t + 1 < N_TF)
            def _():
                dy_cp(t + 1, 1 - slot).start()

            dy_cp(t, slot).wait()
            dyt = dystage[slot].astype(jnp.bfloat16)
            res = lax.dot_general(
                xall[:, :], dyt,
                dimension_numbers=(((0,), (0,)), ((), ())),
                preferred_element_type=jnp.float32,
            )
            ci, cc = t // TPC, (t % TPC) * TF

            @pl.when(t >= TPC)
            def _():
                e_buf[ci, :, pl.ds(cc, TF)] = res[0:Q, :].astype(jnp.bfloat16)

            d_buf[:, pl.ds(t * TF, TF)] = res[Q:, :].astype(jnp.bfloat16)

            if COMM:
                @pl.when(t % TPC == TPC - 1)
                def _():
                    chunk_end(ci)

            return carry

        lax.fori_loop(0, N_TF, compute_step, 0)

        if COMM:
            for i in range(NC - L1, NC):
                y_rdma(i).wait_recv()
                z_rdma(i).start()
            for i in range(NC - L2, NC):
                z_rdma(i).wait_recv()
                if i > 0:
                    out_wait(i - 1)
                emit_out(i)
        else:
            for i in range(NC):
                if i > 0:
                    out_wait(i - 1)
                emit_out(i)
        out_wait(NC - 1)

        if COMM:
            def drain_step(i, carry):
                y_rdma(i).wait_send()
                z_rdma(i).wait_send()
                return carry

            lax.fori_loop(0, NC, drain_step, 0)

    return pl.pallas_call(
        body,
        out_shape=jax.ShapeDtypeStruct((D_DIM // 2, F_DIM), jnp.bfloat16),
        in_specs=[
            pl.BlockSpec(memory_space=pl.ANY),
            pl.BlockSpec(memory_space=pl.ANY),
        ],
        out_specs=pl.BlockSpec(memory_space=pl.ANY),
        scratch_shapes=[
            pltpu.VMEM((2, M_PER, XC), jnp.float32),
            pltpu.VMEM((M_PER, 3 * Q), jnp.bfloat16),
            pltpu.VMEM((2, M_PER, TF), jnp.float32),
            pltpu.VMEM((NC, Q, CF), jnp.bfloat16),
            pltpu.VMEM((2 * Q, F_DIM), jnp.bfloat16),
            pltpu.VMEM((NC, Q, CF), jnp.bfloat16),
            pltpu.VMEM((NC, Q, CF), jnp.bfloat16),
            pltpu.VMEM((2 * Q, CF), jnp.bfloat16),
            pltpu.SemaphoreType.DMA((4,)),
            pltpu.SemaphoreType.DMA,
            pltpu.SemaphoreType.DMA((NC,)),
            pltpu.SemaphoreType.DMA((NC,)),
            pltpu.SemaphoreType.DMA((NC,)),
            pltpu.SemaphoreType.DMA((NC,)),
        ],
        compiler_params=pltpu.CompilerParams(
            collective_id=0 if COMM else None,
            vmem_limit_bytes=64 * 1024 * 1024,
        ),
    )(x, dy)


# device time: 124742 ns/iter; 1.3522x vs baseline; 1.0982x over previous
import jax
import jax.numpy as jnp
from jax import lax
from jax.experimental import pallas as pl
from jax.experimental.pallas import tpu as pltpu

M_PER = 2048
D_DIM = 2048
F_DIM = 8192
Q = 512
TF = 256
N_TF = F_DIM // TF
XC = 256
NC = 32
CF = F_DIM // NC
TPC = CF // TF
COMM = True


def kernel(x, dy):
    def body(x_hbm, dy_hbm, out_hbm,
             xstage, xall, dystage, e_buf, d_buf, r1_buf, r2_buf, out_stage,
             local_sems, out_sem, y_send_sems, y_recv_sems,
             z_send_sems, z_recv_sems):
        my_x = lax.axis_index("x")
        my_y = lax.axis_index("y")
        my_z = lax.axis_index("z")
        y_nbr = (my_x, 1 - my_y, my_z)
        z_nbr = (my_x, my_y, 1 - my_z)

        if COMM:
            barrier_sem = pltpu.get_barrier_semaphore()
            for nbr in (y_nbr, z_nbr):
                pl.semaphore_signal(barrier_sem, inc=1, device_id=nbr,
                                    device_id_type=pl.DeviceIdType.MESH)
            pl.semaphore_wait(barrier_sem, 2)

        def y_rdma(i):
            return pltpu.make_async_remote_copy(
                src_ref=e_buf.at[i], dst_ref=r1_buf.at[i],
                send_sem=y_send_sems.at[i], recv_sem=y_recv_sems.at[i],
                device_id=y_nbr, device_id_type=pl.DeviceIdType.MESH)

        def z_rdma(i):
            return pltpu.make_async_remote_copy(
                src_ref=r1_buf.at[i], dst_ref=r2_buf.at[i],
                send_sem=z_send_sems.at[i], recv_sem=z_recv_sems.at[i],
                device_id=z_nbr, device_id_type=pl.DeviceIdType.MESH)

        qe_col = (1 - my_y) * 1024 + my_z * Q
        qd_col = my_y * 1024
        xoffs = []
        for i, coloff in enumerate([qe_col, qd_col, qd_col + Q]):
            for c in range(Q // XC):
                xoffs.append((i * Q + c * XC, coloff + c * XC))
        cps = [None, None]
        for j, (dst_off, src_off) in enumerate(xoffs):
            slot = j % 2
            cps[slot] = pltpu.make_async_copy(
                x_hbm.at[:, pl.ds(src_off, XC)], xstage.at[slot],
                local_sems.at[slot])
            cps[slot].start()
            if j > 0:
                prev_dst = xoffs[j - 1][0]
                cps[(j - 1) % 2].wait()
                xall[:, prev_dst:prev_dst + XC] = (
                    xstage[(j - 1) % 2].astype(jnp.bfloat16))
        cps[(len(xoffs) - 1) % 2].wait()
        last_dst = xoffs[-1][0]
        xall[:, last_dst:last_dst + XC] = (
            xstage[(len(xoffs) - 1) % 2].astype(jnp.bfloat16))

        def dy_cp(t, slot):
            return pltpu.make_async_copy(
                dy_hbm.at[:, pl.ds(t * TF, TF)], dystage.at[slot],
                local_sems.at[slot])

        dy_cp(0, 0).start()

        z_is_0 = (my_z == 0)

        def emit_out(i):
            d = d_buf[:, pl.ds(i * CF, CF)].astype(jnp.float32)
            r1 = r1_buf[i].astype(jnp.float32)
            r2 = r2_buf[i].astype(jnp.float32)
            out_stage[0:Q, :] = (
                d[0:Q, :] + jnp.where(z_is_0, r1, r2)).astype(jnp.bfloat16)
            out_stage[Q:, :] = (
                d[Q:, :] + jnp.where(z_is_0, r2, r1)).astype(jnp.bfloat16)
            pltpu.make_async_copy(
                out_stage, out_hbm.at[:, pl.ds(i * CF, CF)], out_sem).start()

        def out_wait(i):
            pltpu.make_async_copy(
                out_stage, out_hbm.at[:, pl.ds(i * CF, CF)], out_sem).wait()

        L1 = 2
        L2 = 4

        def chunk_end(ci):
            y_rdma(ci).start()

            @pl.when(ci >= L1)
            def _():
                cz = ci - L1
                y_rdma(cz).wait_recv()
                z_rdma(cz).start()

            @pl.when(ci >= L2)
            def _():
                ce = ci - L2
                z_rdma(ce).wait_recv()

                @pl.when(ce > 0)
                def _():
                    out_wait(ce - 1)

                emit_out(ce)

        def compute_step(t, carry):
            slot = t % 2

            @pl.when(t + 1 < N_TF)
            def _():
                dy_cp(t + 1, 1 - slot).start()

            dy_cp(t, slot).wait()
            dyt = dystage[slot].astype(jnp.bfloat16)
            res = lax.dot_general(
                xall[:, :], dyt,
                dimension_numbers=(((0,), (0,)), ((), ())),
                preferred_element_type=jnp.float32,
            )
            ci, cc = t // TPC, (t % TPC) * TF
            e_buf[ci, :, pl.ds(cc, TF)] = res[0:Q, :].astype(jnp.bfloat16)
            d_buf[:, pl.ds(t * TF, TF)] = res[Q:, :].astype(jnp.bfloat16)

            if COMM:
                @pl.when(t % TPC == TPC - 1)
                def _():
                    chunk_end(ci)

            return carry

        lax.fori_loop(0, N_TF, compute_step, 0)

        if COMM:
            for i in range(NC - L1, NC):
                y_rdma(i).wait_recv()
                z_rdma(i).start()
            for i in range(NC - L2, NC):
                z_rdma(i).wait_recv()
                if i > 0:
                    out_wait(i - 1)
                emit_out(i)
        else:
            for i in range(NC):
                if i > 0:
                    out_wait(i - 1)
                emit_out(i)
        out_wait(NC - 1)

        if COMM:
            def drain_step(i, carry):
                y_rdma(i).wait_send()
                z_rdma(i).wait_send()
                return carry

            lax.fori_loop(0, NC, drain_step, 0)

    return pl.pallas_call(
        body,
        out_shape=jax.ShapeDtypeStruct((D_DIM // 2, F_DIM), jnp.bfloat16),
        in_specs=[
            pl.BlockSpec(memory_space=pl.ANY),
            pl.BlockSpec(memory_space=pl.ANY),
        ],
        out_specs=pl.BlockSpec(memory_space=pl.ANY),
        scratch_shapes=[
            pltpu.VMEM((2, M_PER, XC), jnp.float32),
            pltpu.VMEM((M_PER, 3 * Q), jnp.bfloat16),
            pltpu.VMEM((2, M_PER, TF), jnp.float32),
            pltpu.VMEM((NC, Q, CF), jnp.bfloat16),
            pltpu.VMEM((2 * Q, F_DIM), jnp.bfloat16),
            pltpu.VMEM((NC, Q, CF), jnp.bfloat16),
            pltpu.VMEM((NC, Q, CF), jnp.bfloat16),
            pltpu.VMEM((2 * Q, CF), jnp.bfloat16),
            pltpu.SemaphoreType.DMA((2,)),
            pltpu.SemaphoreType.DMA,
            pltpu.SemaphoreType.DMA((NC,)),
            pltpu.SemaphoreType.DMA((NC,)),
            pltpu.SemaphoreType.DMA((NC,)),
            pltpu.SemaphoreType.DMA((NC,)),
        ],
        compiler_params=pltpu.CompilerParams(
            collective_id=0 if COMM else None,
            vmem_limit_bytes=64 * 1024 * 1024,
        ),
    )(x, dy)
